# baseline (speedup 1.0000x reference)
import functools

import jax
import jax.numpy as jnp
from jax import lax
from jax.experimental import pallas as pl
from jax.experimental.pallas import tpu as pltpu
from jax.experimental.pallas import tpu_sc as plsc

B = 4
N = 65536
M = N * 3
GRP = 8
CH = M // GRP
PCH = N // GRP
L = 16
NPV = PCH // L
UN = 8
EPS = 1e-07
_MASK31 = 0x7FFFFFFF


def _sc_point_loss(pred, target, weight):
    mesh = plsc.VectorSubcoreMesh(core_axis_name="c", subcore_axis_name="s")

    @functools.partial(
        pl.kernel,
        mesh=mesh,
        out_type=jax.ShapeDtypeStruct((B * L,), jnp.float32),
        compiler_params=pltpu.CompilerParams(needs_layout_passes=False),
        scratch_types=[
            pltpu.VMEM((PCH,), jnp.float32),
            pltpu.VMEM((PCH,), jnp.float32),
            pltpu.VMEM((PCH,), jnp.float32),
            pltpu.VMEM((PCH,), jnp.float32),
            pltpu.VMEM((PCH,), jnp.float32),
            pltpu.VMEM((PCH,), jnp.float32),
            pltpu.VMEM((PCH,), jnp.float32),
            pltpu.VMEM((CH + UN * L,), jnp.int32),
            pltpu.VMEM((CH + UN * L,), jnp.float32),
            pltpu.VMEM((L,), jnp.float32),
            pltpu.VMEM((GRP * L,), jnp.float32),
            pltpu.VMEM((L,), jnp.float32),
            pltpu.VMEM_SHARED((2, GRP * L), jnp.float32),
        ],
    )
    def k(pred_in, target_in, weight_hbm, out_hbm,
          p0_v, p1_v, p2_v, t0_v, t1_v, t2_v,
          w_v, key_v, wx_v, stage_v, grp_v, out_v, shared):
        cid = lax.axis_index("c")
        sid = lax.axis_index("s")
        g = sid // GRP
        lid = sid % GRP
        b = cid * 2 + g
        lane = lax.iota(jnp.int32, L)

        r0 = b * N + lid * PCH
        p_bufs = (p0_v, p1_v, p2_v)
        t_bufs = (t0_v, t1_v, t2_v)
        for kc in range(3):
            pltpu.sync_copy(pred_in.at[b * 3 + kc, pl.ds(lid * PCH, PCH)],
                            p_bufs[kc])
            pltpu.sync_copy(target_in.at[b * 3 + kc, pl.ds(lid * PCH, PCH)],
                            t_bufs[kc])
        pltpu.sync_copy(weight_hbm.at[b, pl.ds(lid * PCH, PCH)], w_v)

        zero = jnp.zeros((L,), jnp.float32)
        eps = jnp.float32(EPS)

        def global_vec(vec):
            stage_v[...] = vec
            plsc.subcore_barrier()
            pltpu.sync_copy(stage_v, shared.at[g, pl.ds(lid * L, L)])
            plsc.subcore_barrier()
            pltpu.sync_copy(shared.at[g], grp_v)

            def rd(j, acc):
                return acc + grp_v[pl.ds(j * L, L)]

            return lax.fori_loop(0, GRP, rd, zero)

        def global_sum(vec):
            return jnp.sum(global_vec(vec))

        def lane_pick(gv, j):
            return jnp.sum(jnp.where(lane == j, gv, jnp.float32(0.0)))

        def global_sum3(v1, v2, v3):
            s1 = jnp.sum(v1)
            s2 = jnp.sum(v2)
            s3 = jnp.sum(v3)
            packed = jnp.where(
                lane == 0, s1,
                jnp.where(lane == 1, s2,
                          jnp.where(lane == 2, s3, jnp.float32(0.0))))
            gv = global_vec(packed)
            return lane_pick(gv, 0), lane_pick(gv, 1), lane_pick(gv, 2)

        def make_pass_a(kc):
            pb, tb = p_bufs[kc], t_bufs[kc]

            def pass_a(i, acc):
                for u in range(UN):
                    v = i * UN + u
                    sl = pl.ds(v * L, L)
                    p = pb[sl]
                    t = tb[sl]
                    w = w_v[sl]
                    sgn = jnp.where(
                        p >= 0.0, jnp.float32(1.0), jnp.float32(-1.0))
                    xa = jnp.abs(p)
                    ya = t * sgn
                    ratio = ya / jnp.maximum(xa, eps)
                    bits = plsc.bitcast(ratio, jnp.int32)
                    key = jnp.where(
                        bits >= 0, bits, bits ^ jnp.int32(_MASK31))
                    so = pl.ds(kc * PCH + v * L, L)
                    key_v[so] = key
                    wx_v[so] = xa * w
                    acc = acc + xa * w
                return acc

            return pass_a

        tvec = zero
        for kc in range(3):
            tvec = lax.fori_loop(0, NPV // UN, make_pass_a(kc), tvec)
        t_half = global_sum(tvec) * jnp.float32(0.5)

        def count_lt(q):
            qv = jnp.full((L,), q, jnp.int32)

            def body(i, acc):
                for u in range(UN):
                    sl = pl.ds((i * UN + u) * L, L)
                    kk = key_v[sl]
                    vv = wx_v[sl]
                    acc = acc + jnp.where(kk < qv, vv, jnp.float32(0.0))
                return acc

            return lax.fori_loop(0, CH // (UN * L), body, zero)

        c0 = global_sum(count_lt(jnp.int32(0)))
        acc0 = c0 < t_half
        p_key = jnp.where(acc0, jnp.int32(0), jnp.int32(-2147483648))
        f_p = jnp.where(acc0, c0, jnp.float32(0.0))

        def count3(q1, q2, q3):
            qv1 = jnp.full((L,), q1, jnp.int32)
            qv2 = jnp.full((L,), q2, jnp.int32)
            qv3 = jnp.full((L,), q3, jnp.int32)

            def body(i, accs):
                a1, a2, a3 = accs
                for u in range(UN):
                    sl = pl.ds((i * UN + u) * L, L)
                    kk = key_v[sl]
                    vv = wx_v[sl]
                    a1 = a1 + jnp.where(kk < qv1, vv, jnp.float32(0.0))
                    a2 = a2 + jnp.where(kk < qv2, vv, jnp.float32(0.0))
                    a3 = a3 + jnp.where(kk < qv3, vv, jnp.float32(0.0))
                return (a1, a2, a3)

            return lax.fori_loop(0, CH // (UN * L), body, (zero, zero, zero))

        def radix4_step(p_key, f_p, c1, c2, c3, step):
            t_star = ((c1 < t_half).astype(jnp.int32)
                      + (c2 < t_half).astype(jnp.int32)
                      + (c3 < t_half).astype(jnp.int32))
            p_new = p_key + t_star * step
            f_new = jnp.where(
                c3 < t_half, c3,
                jnp.where(c2 < t_half, c2,
                          jnp.where(c1 < t_half, c1, f_p)))
            return p_new, f_new

        def round_body(r, carry):
            p_key, f_p = carry
            step = jnp.int32(1) << (29 - 2 * r)
            a1, a2, a3 = count3(p_key + step, p_key + 2 * step,
                                p_key + 3 * step)
            c1, c2, c3 = global_sum3(a1, a2, a3)
            return radix4_step(p_key, f_p, c1, c2, c3, step)

        p_key, f_p = lax.fori_loop(0, 4, round_body, (p_key, f_p))

        pkv = jnp.full((L,), p_key, jnp.int32)
        lim = jnp.int32(1 << 23)

        def compact(i, off):
            sl = pl.ds(i * L, L)
            kk = key_v[sl]
            vv = wx_v[sl]
            mask = (kk >= pkv) & ((kk - pkv) < lim)
            plsc.store_compressed(key_v.at[pl.ds(off, L)], kk, mask=mask)
            plsc.store_compressed(wx_v.at[pl.ds(off, L)], vv, mask=mask)
            return off + jnp.max(plsc.all_reduce_population_count(mask))

        cnt = lax.fori_loop(0, CH // L, compact, jnp.int32(0))
        for j in range(UN):
            slp = pl.ds(cnt + j * L, L)
            key_v[slp] = jnp.full((L,), jnp.int32(_MASK31), jnp.int32)
            wx_v[slp] = zero
        nblk = (cnt + (UN * L - 1)) // (UN * L)

        def count_lt2(q):
            qv = jnp.full((L,), q, jnp.int32)

            def body(i, acc):
                for u in range(UN):
                    sl = pl.ds((i * UN + u) * L, L)
                    acc = acc + jnp.where(
                        key_v[sl] < qv, wx_v[sl], jnp.float32(0.0))
                return acc

            return lax.fori_loop(0, nblk, body, zero)

        def count3s(q1, q2, q3):
            qv1 = jnp.full((L,), q1, jnp.int32)
            qv2 = jnp.full((L,), q2, jnp.int32)
            qv3 = jnp.full((L,), q3, jnp.int32)

            def body(i, accs):
                a1, a2, a3 = accs
                for u in range(UN):
                    sl = pl.ds((i * UN + u) * L, L)
                    kk = key_v[sl]
                    vv = wx_v[sl]
                    a1 = a1 + jnp.where(kk < qv1, vv, jnp.float32(0.0))
                    a2 = a2 + jnp.where(kk < qv2, vv, jnp.float32(0.0))
                    a3 = a3 + jnp.where(kk < qv3, vv, jnp.float32(0.0))
                return (a1, a2, a3)

            return lax.fori_loop(0, nblk, body, (zero, zero, zero))

        def round2_body(r, carry):
            p_key, f_p = carry
            step = jnp.int32(1) << (21 - 2 * r)
            a1, a2, a3 = count3s(p_key + step, p_key + 2 * step,
                                 p_key + 3 * step)
            c1, c2, c3 = global_sum3(a1, a2, a3)
            c1 = c1 + f_p
            c2 = c2 + f_p
            c3 = c3 + f_p
            return radix4_step(p_key, f_p, c1, c2, c3, step)

        p_key, f_p = lax.fori_loop(0, 11, round2_body, (p_key, f_p))

        q = p_key + jnp.int32(1)
        c = f_p + global_sum(count_lt2(q))
        p_key = jnp.where(c < t_half, q, p_key)

        pbits = jnp.where(p_key >= 0, p_key, p_key ^ jnp.int32(_MASK31))
        a_vec = plsc.bitcast(jnp.full((L,), pbits, jnp.int32), jnp.float32)

        def make_pass_c(kc):
            pb, tb = p_bufs[kc], t_bufs[kc]

            def pass_c(i, acc):
                for u in range(UN):
                    sl = pl.ds((i * UN + u) * L, L)
                    p = pb[sl]
                    t = tb[sl]
                    w = w_v[sl]
                    acc = acc + w * jnp.abs(a_vec * p - t)
                return acc

            return pass_c

        num_vec = zero
        for kc in range(3):
            num_vec = lax.fori_loop(0, NPV // UN, make_pass_c(kc), num_vec)

        def pass_w(i, acc):
            return acc + w_v[pl.ds(i * L, L)]

        den_vec = lax.fori_loop(0, PCH // L, pass_w, zero)

        sn = jnp.sum(num_vec)
        sd = jnp.sum(den_vec)
        gv = global_vec(jnp.where(lane == 0, sn,
                                  jnp.where(lane == 1, sd,
                                            jnp.float32(0.0))))
        num = lane_pick(gv, 0)
        den = lane_pick(gv, 1)

        @pl.when(lid == 0)
        def _():
            out_v[...] = jnp.where(
                lane == 0, num, jnp.where(lane == 1, den, jnp.float32(0.0)))
            pltpu.sync_copy(out_v, out_hbm.at[pl.ds(b * L, L)])

    return k(pred, target, weight)


def kernel(pred, target, weight):
    pred_t = jnp.swapaxes(pred, 1, 2).reshape(B * 3, N)
    target_t = jnp.swapaxes(target, 1, 2).reshape(B * 3, N)
    out = _sc_point_loss(pred_t, target_t, weight).reshape(B, L)
    per_batch = out[:, 0]
    denom = 3.0 * jnp.maximum(out[:, 1], EPS)
    return jnp.mean(per_batch / denom)


# speedup vs baseline: 1.1053x; 1.1053x over previous
import functools

import jax
import jax.numpy as jnp
from jax import lax
from jax.experimental import pallas as pl
from jax.experimental.pallas import tpu as pltpu
from jax.experimental.pallas import tpu_sc as plsc

B = 4
N = 65536
M = N * 3
GRP = 8
CH = M // GRP
PCH = N // GRP
L = 16
NPV = PCH // L
UN = 8
EPS = 1e-07
_MASK31 = 0x7FFFFFFF


def _sc_point_loss(pred, target, weight):
    mesh = plsc.VectorSubcoreMesh(core_axis_name="c", subcore_axis_name="s")

    @functools.partial(
        pl.kernel,
        mesh=mesh,
        out_type=jax.ShapeDtypeStruct((B * L,), jnp.float32),
        compiler_params=pltpu.CompilerParams(needs_layout_passes=False),
        scratch_types=[
            pltpu.VMEM((PCH,), jnp.float32),
            pltpu.VMEM((PCH,), jnp.float32),
            pltpu.VMEM((PCH,), jnp.float32),
            pltpu.VMEM((PCH,), jnp.float32),
            pltpu.VMEM((PCH,), jnp.float32),
            pltpu.VMEM((PCH,), jnp.float32),
            pltpu.VMEM((PCH,), jnp.float32),
            pltpu.VMEM((CH + UN * L,), jnp.int32),
            pltpu.VMEM((CH + UN * L,), jnp.float32),
            pltpu.VMEM((L,), jnp.float32),
            pltpu.VMEM((GRP * L,), jnp.float32),
            pltpu.VMEM((L,), jnp.float32),
            pltpu.VMEM_SHARED((2, GRP * L), jnp.float32),
        ],
    )
    def k(pred_in, target_in, weight_hbm, out_hbm,
          p0_v, p1_v, p2_v, t0_v, t1_v, t2_v,
          w_v, key_v, wx_v, stage_v, grp_v, out_v, shared):
        cid = lax.axis_index("c")
        sid = lax.axis_index("s")
        g = sid // GRP
        lid = sid % GRP
        b = cid * 2 + g
        lane = lax.iota(jnp.int32, L)

        r0 = b * N + lid * PCH
        p_bufs = (p0_v, p1_v, p2_v)
        t_bufs = (t0_v, t1_v, t2_v)
        for kc in range(3):
            pltpu.sync_copy(pred_in.at[b * 3 + kc, pl.ds(lid * PCH, PCH)],
                            p_bufs[kc])
            pltpu.sync_copy(target_in.at[b * 3 + kc, pl.ds(lid * PCH, PCH)],
                            t_bufs[kc])
        pltpu.sync_copy(weight_hbm.at[b, pl.ds(lid * PCH, PCH)], w_v)

        zero = jnp.zeros((L,), jnp.float32)
        eps = jnp.float32(EPS)

        def global_sum(vec):
            stage_v[...] = vec
            plsc.subcore_barrier()
            pltpu.sync_copy(stage_v, shared.at[g, pl.ds(lid * L, L)])
            plsc.subcore_barrier()
            pltpu.sync_copy(shared.at[g], grp_v)

            def rd(j, acc):
                return acc + grp_v[pl.ds(j * L, L)]

            return jnp.sum(lax.fori_loop(0, GRP, rd, zero))

        def make_pass_a(kc):
            pb, tb = p_bufs[kc], t_bufs[kc]

            def pass_a(i, acc):
                for u in range(UN):
                    v = i * UN + u
                    sl = pl.ds(v * L, L)
                    p = pb[sl]
                    t = tb[sl]
                    w = w_v[sl]
                    sgn = jnp.where(
                        p >= 0.0, jnp.float32(1.0), jnp.float32(-1.0))
                    xa = jnp.abs(p)
                    ya = t * sgn
                    ratio = ya / jnp.maximum(xa, eps)
                    bits = plsc.bitcast(ratio, jnp.int32)
                    key = jnp.where(
                        bits >= 0, bits, bits ^ jnp.int32(_MASK31))
                    so = pl.ds(kc * PCH + v * L, L)
                    key_v[so] = key
                    wx_v[so] = xa * w
                    acc = acc + xa * w
                return acc

            return pass_a

        tvec = zero
        for kc in range(3):
            tvec = lax.fori_loop(0, NPV // UN, make_pass_a(kc), tvec)
        t_half = global_sum(tvec) * jnp.float32(0.5)

        def count_lt(q):
            qv = jnp.full((L,), q, jnp.int32)

            def body(i, acc):
                for u in range(UN):
                    sl = pl.ds((i * UN + u) * L, L)
                    kk = key_v[sl]
                    vv = wx_v[sl]
                    acc = acc + jnp.where(kk < qv, vv, jnp.float32(0.0))
                return acc

            return lax.fori_loop(0, CH // (UN * L), body, zero)

        c0 = global_sum(count_lt(jnp.int32(0)))
        acc0 = c0 < t_half
        p_key = jnp.where(acc0, jnp.int32(0), jnp.int32(-2147483648))
        f_p = jnp.where(acc0, c0, jnp.float32(0.0))

        def round_body(r, carry):
            p_key, f_p = carry
            q = p_key + (jnp.int32(1) << (30 - r))
            c = global_sum(count_lt(q))
            acc = c < t_half
            return (jnp.where(acc, q, p_key), jnp.where(acc, c, f_p))

        p_key, f_p = lax.fori_loop(0, 7, round_body, (p_key, f_p))

        pkv = jnp.full((L,), p_key, jnp.int32)
        lim = jnp.int32(1 << 24)

        def compact(i, off):
            sl = pl.ds(i * L, L)
            kk = key_v[sl]
            vv = wx_v[sl]
            mask = (kk >= pkv) & ((kk - pkv) < lim)
            plsc.store_compressed(key_v.at[pl.ds(off, L)], kk, mask=mask)
            plsc.store_compressed(wx_v.at[pl.ds(off, L)], vv, mask=mask)
            return off + jnp.max(plsc.all_reduce_population_count(mask))

        cnt = lax.fori_loop(0, CH // L, compact, jnp.int32(0))
        for j in range(UN):
            slp = pl.ds(cnt + j * L, L)
            key_v[slp] = jnp.full((L,), jnp.int32(_MASK31), jnp.int32)
            wx_v[slp] = zero
        nblk = (cnt + (UN * L - 1)) // (UN * L)

        def count_lt2(q):
            qv = jnp.full((L,), q, jnp.int32)

            def body(i, acc):
                for u in range(UN):
                    sl = pl.ds((i * UN + u) * L, L)
                    acc = acc + jnp.where(
                        key_v[sl] < qv, wx_v[sl], jnp.float32(0.0))
                return acc

            return lax.fori_loop(0, nblk, body, zero)

        def round2_body(r, p_key):
            q = p_key + (jnp.int32(1) << (23 - r))
            c = f_p + global_sum(count_lt2(q))
            return jnp.where(c < t_half, q, p_key)

        p_key = lax.fori_loop(0, 24, round2_body, p_key)

        pbits = jnp.where(p_key >= 0, p_key, p_key ^ jnp.int32(_MASK31))
        a_vec = plsc.bitcast(jnp.full((L,), pbits, jnp.int32), jnp.float32)

        def make_pass_c(kc):
            pb, tb = p_bufs[kc], t_bufs[kc]

            def pass_c(i, acc):
                for u in range(UN):
                    sl = pl.ds((i * UN + u) * L, L)
                    p = pb[sl]
                    t = tb[sl]
                    w = w_v[sl]
                    acc = acc + w * jnp.abs(a_vec * p - t)
                return acc

            return pass_c

        num_vec = zero
        for kc in range(3):
            num_vec = lax.fori_loop(0, NPV // UN, make_pass_c(kc), num_vec)

        def pass_w(i, acc):
            return acc + w_v[pl.ds(i * L, L)]

        den_vec = lax.fori_loop(0, PCH // L, pass_w, zero)

        sn = jnp.sum(num_vec)
        sd = jnp.sum(den_vec)
        stage_v[...] = jnp.where(lane == 0, sn,
                                 jnp.where(lane == 1, sd, jnp.float32(0.0)))
        plsc.subcore_barrier()
        pltpu.sync_copy(stage_v, shared.at[g, pl.ds(lid * L, L)])
        plsc.subcore_barrier()
        pltpu.sync_copy(shared.at[g], grp_v)

        def rd8(j, acc):
            return acc + grp_v[pl.ds(j * L, L)]

        gv = lax.fori_loop(0, GRP, rd8, zero)
        num = jnp.sum(jnp.where(lane == 0, gv, jnp.float32(0.0)))
        den = jnp.sum(jnp.where(lane == 1, gv, jnp.float32(0.0)))

        @pl.when(lid == 0)
        def _():
            out_v[...] = jnp.where(
                lane == 0, num, jnp.where(lane == 1, den, jnp.float32(0.0)))
            pltpu.sync_copy(out_v, out_hbm.at[pl.ds(b * L, L)])

    return k(pred, target, weight)


def kernel(pred, target, weight):
    pred_t = jnp.swapaxes(pred, 1, 2).reshape(B * 3, N)
    target_t = jnp.swapaxes(target, 1, 2).reshape(B * 3, N)
    out = _sc_point_loss(pred_t, target_t, weight).reshape(B, L)
    per_batch = out[:, 0]
    denom = 3.0 * jnp.maximum(out[:, 1], EPS)
    return jnp.mean(per_batch / denom)


# speedup vs baseline: 1.1586x; 1.0482x over previous
import functools

import jax
import jax.numpy as jnp
from jax import lax
from jax.experimental import pallas as pl
from jax.experimental.pallas import tpu as pltpu
from jax.experimental.pallas import tpu_sc as plsc

B = 4
N = 65536
M = N * 3
GRP = 8
CH = M // GRP
PCH = N // GRP
L = 16
NPV = PCH // L
UN = 8
EPS = 1e-07
_MASK31 = 0x7FFFFFFF


def _sc_point_loss(pred, target, weight):
    mesh = plsc.VectorSubcoreMesh(core_axis_name="c", subcore_axis_name="s")

    @functools.partial(
        pl.kernel,
        mesh=mesh,
        out_type=jax.ShapeDtypeStruct((B * L,), jnp.float32),
        compiler_params=pltpu.CompilerParams(needs_layout_passes=False),
        scratch_types=[
            pltpu.VMEM((PCH,), jnp.float32),
            pltpu.VMEM((PCH,), jnp.float32),
            pltpu.VMEM((PCH,), jnp.float32),
            pltpu.VMEM((PCH,), jnp.float32),
            pltpu.VMEM((PCH,), jnp.float32),
            pltpu.VMEM((PCH,), jnp.float32),
            pltpu.VMEM((PCH,), jnp.float32),
            pltpu.VMEM((CH + UN * L,), jnp.int32),
            pltpu.VMEM((CH + UN * L,), jnp.float32),
            pltpu.VMEM((L,), jnp.float32),
            pltpu.VMEM((GRP * L,), jnp.float32),
            pltpu.VMEM((L,), jnp.float32),
            pltpu.VMEM_SHARED((2, GRP * L), jnp.float32),
            pltpu.SemaphoreType.DMA,
        ],
    )
    def k(pred_in, target_in, weight_hbm, out_hbm,
          p0_v, p1_v, p2_v, t0_v, t1_v, t2_v,
          w_v, key_v, wx_v, stage_v, grp_v, out_v, shared, dsem):
        cid = lax.axis_index("c")
        sid = lax.axis_index("s")
        g = sid // GRP
        lid = sid % GRP
        b = cid * 2 + g
        lane = lax.iota(jnp.int32, L)

        r0 = b * N + lid * PCH
        p_bufs = (p0_v, p1_v, p2_v)
        t_bufs = (t0_v, t1_v, t2_v)
        copies = []
        for kc in range(3):
            copies.append(pltpu.async_copy(
                pred_in.at[b * 3 + kc, pl.ds(lid * PCH, PCH)],
                p_bufs[kc], dsem))
            copies.append(pltpu.async_copy(
                target_in.at[b * 3 + kc, pl.ds(lid * PCH, PCH)],
                t_bufs[kc], dsem))
        copies.append(pltpu.async_copy(
            weight_hbm.at[b, pl.ds(lid * PCH, PCH)], w_v, dsem))
        for cp in copies:
            cp.wait()

        zero = jnp.zeros((L,), jnp.float32)
        eps = jnp.float32(EPS)

        def global_sum(vec):
            stage_v[...] = vec
            plsc.subcore_barrier()
            pltpu.sync_copy(stage_v, shared.at[g, pl.ds(lid * L, L)])
            plsc.subcore_barrier()
            pltpu.sync_copy(shared.at[g], grp_v)

            def rd(j, acc):
                return acc + grp_v[pl.ds(j * L, L)]

            return jnp.sum(lax.fori_loop(0, GRP, rd, zero))

        def make_pass_a(kc):
            pb, tb = p_bufs[kc], t_bufs[kc]

            def pass_a(i, acc):
                for u in range(UN):
                    v = i * UN + u
                    sl = pl.ds(v * L, L)
                    p = pb[sl]
                    t = tb[sl]
                    w = w_v[sl]
                    sgn = jnp.where(
                        p >= 0.0, jnp.float32(1.0), jnp.float32(-1.0))
                    xa = jnp.abs(p)
                    ya = t * sgn
                    ratio = ya / jnp.maximum(xa, eps)
                    bits = plsc.bitcast(ratio, jnp.int32)
                    key = jnp.where(
                        bits >= 0, bits, bits ^ jnp.int32(_MASK31))
                    so = pl.ds(kc * PCH + v * L, L)
                    key_v[so] = key
                    wx_v[so] = xa * w
                    acc = acc + xa * w
                return acc

            return pass_a

        tvec = zero
        for kc in range(3):
            tvec = lax.fori_loop(0, NPV // UN, make_pass_a(kc), tvec)
        t_half = global_sum(tvec) * jnp.float32(0.5)

        def count_lt(q):
            qv = jnp.full((L,), q, jnp.int32)

            def body(i, acc):
                for u in range(UN):
                    sl = pl.ds((i * UN + u) * L, L)
                    kk = key_v[sl]
                    vv = wx_v[sl]
                    acc = acc + jnp.where(kk < qv, vv, jnp.float32(0.0))
                return acc

            return lax.fori_loop(0, CH // (UN * L), body, zero)

        c0 = global_sum(count_lt(jnp.int32(0)))
        acc0 = c0 < t_half
        p_key = jnp.where(acc0, jnp.int32(0), jnp.int32(-2147483648))
        f_p = jnp.where(acc0, c0, jnp.float32(0.0))

        def round_body(r, carry):
            p_key, f_p = carry
            q = p_key + (jnp.int32(1) << (30 - r))
            c = global_sum(count_lt(q))
            acc = c < t_half
            return (jnp.where(acc, q, p_key), jnp.where(acc, c, f_p))

        p_key, f_p = lax.fori_loop(0, 7, round_body, (p_key, f_p))

        pkv = jnp.full((L,), p_key, jnp.int32)
        lim = jnp.int32(1 << 24)

        def compact(i, off):
            sl = pl.ds(i * L, L)
            kk = key_v[sl]
            vv = wx_v[sl]
            mask = (kk >= pkv) & ((kk - pkv) < lim)
            plsc.store_compressed(key_v.at[pl.ds(off, L)], kk, mask=mask)
            plsc.store_compressed(wx_v.at[pl.ds(off, L)], vv, mask=mask)
            return off + jnp.max(plsc.all_reduce_population_count(mask))

        cnt = lax.fori_loop(0, CH // L, compact, jnp.int32(0))
        for j in range(UN):
            slp = pl.ds(cnt + j * L, L)
            key_v[slp] = jnp.full((L,), jnp.int32(_MASK31), jnp.int32)
            wx_v[slp] = zero
        nblk = (cnt + (UN * L - 1)) // (UN * L)

        def count_lt2(q):
            qv = jnp.full((L,), q, jnp.int32)

            def body(i, acc):
                for u in range(UN):
                    sl = pl.ds((i * UN + u) * L, L)
                    acc = acc + jnp.where(
                        key_v[sl] < qv, wx_v[sl], jnp.float32(0.0))
                return acc

            return lax.fori_loop(0, nblk, body, zero)

        def round2_body(r, p_key):
            q = p_key + (jnp.int32(1) << (23 - r))
            c = f_p + global_sum(count_lt2(q))
            return jnp.where(c < t_half, q, p_key)

        p_key = lax.fori_loop(0, 24, round2_body, p_key)

        pbits = jnp.where(p_key >= 0, p_key, p_key ^ jnp.int32(_MASK31))
        a_vec = plsc.bitcast(jnp.full((L,), pbits, jnp.int32), jnp.float32)

        def make_pass_c(kc):
            pb, tb = p_bufs[kc], t_bufs[kc]

            def pass_c(i, acc):
                for u in range(UN):
                    sl = pl.ds((i * UN + u) * L, L)
                    p = pb[sl]
                    t = tb[sl]
                    w = w_v[sl]
                    acc = acc + w * jnp.abs(a_vec * p - t)
                return acc

            return pass_c

        num_vec = zero
        for kc in range(3):
            num_vec = lax.fori_loop(0, NPV // UN, make_pass_c(kc), num_vec)

        def pass_w(i, acc):
            return acc + w_v[pl.ds(i * L, L)]

        den_vec = lax.fori_loop(0, PCH // L, pass_w, zero)

        sn = jnp.sum(num_vec)
        sd = jnp.sum(den_vec)
        stage_v[...] = jnp.where(lane == 0, sn,
                                 jnp.where(lane == 1, sd, jnp.float32(0.0)))
        plsc.subcore_barrier()
        pltpu.sync_copy(stage_v, shared.at[g, pl.ds(lid * L, L)])
        plsc.subcore_barrier()
        pltpu.sync_copy(shared.at[g], grp_v)

        def rd8(j, acc):
            return acc + grp_v[pl.ds(j * L, L)]

        gv = lax.fori_loop(0, GRP, rd8, zero)
        num = jnp.sum(jnp.where(lane == 0, gv, jnp.float32(0.0)))
        den = jnp.sum(jnp.where(lane == 1, gv, jnp.float32(0.0)))

        @pl.when(lid == 0)
        def _():
            out_v[...] = jnp.where(
                lane == 0, num, jnp.where(lane == 1, den, jnp.float32(0.0)))
            pltpu.sync_copy(out_v, out_hbm.at[pl.ds(b * L, L)])

    return k(pred, target, weight)


def kernel(pred, target, weight):
    pred_t = jnp.swapaxes(pred, 1, 2).reshape(B * 3, N)
    target_t = jnp.swapaxes(target, 1, 2).reshape(B * 3, N)
    out = _sc_point_loss(pred_t, target_t, weight).reshape(B, L)
    per_batch = out[:, 0]
    denom = 3.0 * jnp.maximum(out[:, 1], EPS)
    return jnp.mean(per_batch / denom)
